# baseline (speedup 1.0000x reference)
import functools

import jax
import jax.numpy as jnp
from jax import lax
from jax.experimental import pallas as pl
from jax.experimental.pallas import tpu as pltpu
from jax.experimental.pallas import tpu_sc as plsc

D = 64
NC, NS = 2, 16
NW = NC * NS
CH = 256
K = 4


@functools.lru_cache(maxsize=4)
def _make_gather(B, V):
    b_per_w = B // NW
    n_win = b_per_w // (CH * K)
    mesh = plsc.VectorSubcoreMesh(core_axis_name="c", subcore_axis_name="s")

    scratch = [pltpu.VMEM((b_per_w,), jnp.int32)]
    scratch += [pltpu.VMEM((CH, D), jnp.float32) for _ in range(K)]
    scratch += [pltpu.SemaphoreType.DMA for _ in range(2 * K)]

    @functools.partial(
        pl.kernel,
        out_type=jax.ShapeDtypeStruct((B, D), jnp.float32),
        mesh=mesh,
        compiler_params=pltpu.CompilerParams(use_tc_tiling_on_sc=False),
        scratch_types=scratch,
    )
    def gather_kernel(idx_hbm, table_hbm, out_hbm, idx_v, *bufs_and_sems):
        rows = bufs_and_sems[:K]
        gsem = bufs_and_sems[K:2 * K]
        osem = bufs_and_sems[2 * K:]
        wid = lax.axis_index("s") * NC + lax.axis_index("c")
        base = wid * b_per_w
        pltpu.sync_copy(idx_hbm.at[pl.ds(base, b_per_w)], idx_v)

        @pl.loop(0, n_win)
        def _window(w):
            w0 = w * (CH * K)
            gathers = []
            for t in range(K):
                @pl.when(w > 0)
                def _drain():
                    pltpu.make_async_copy(
                        rows[t], out_hbm.at[pl.ds(0, CH)], osem[t]
                    ).wait()

                ids = idx_v.at[pl.ds(w0 + t * CH, CH)]
                gathers.append(
                    pltpu.async_copy(table_hbm.at[ids], rows[t], gsem[t])
                )
            for t in range(K):
                gathers[t].wait()
                pltpu.async_copy(
                    rows[t], out_hbm.at[pl.ds(base + w0 + t * CH, CH)], osem[t]
                )

        for t in range(K):
            pltpu.make_async_copy(
                rows[t], out_hbm.at[pl.ds(0, CH)], osem[t]
            ).wait()

    return gather_kernel


def kernel(x, emb_weight):
    lead_shape = x.shape
    idx = x.reshape(-1).astype(jnp.int32)
    B = idx.shape[0]
    granule = NW * CH * K
    B_pad = ((B + granule - 1) // granule) * granule
    if B_pad != B:
        idx = jnp.pad(idx, (0, B_pad - B))
    out = _make_gather(B_pad, emb_weight.shape[0])(idx, emb_weight)
    if B_pad != B:
        out = out[:B]
    return out.reshape(*lead_shape, D)


# speedup vs baseline: 1.0005x; 1.0005x over previous
import functools

import jax
import jax.numpy as jnp
from jax import lax
from jax.experimental import pallas as pl
from jax.experimental.pallas import tpu as pltpu
from jax.experimental.pallas import tpu_sc as plsc

D = 64
NC, NS = 2, 16
NW = NC * NS
CH = 256
K = 4


@functools.lru_cache(maxsize=4)
def _make_gather(B, V):
    b_per_w = B // NW
    n_win = b_per_w // (CH * K)
    mesh = plsc.VectorSubcoreMesh(core_axis_name="c", subcore_axis_name="s")

    scratch = [pltpu.VMEM((b_per_w,), jnp.int32)]
    scratch += [pltpu.VMEM((CH, D), jnp.float32) for _ in range(K)]
    scratch += [pltpu.SemaphoreType.DMA for _ in range(2 * K)]

    @functools.partial(
        pl.kernel,
        out_type=jax.ShapeDtypeStruct((B, D), jnp.float32),
        mesh=mesh,
        compiler_params=pltpu.CompilerParams(use_tc_tiling_on_sc=False),
        scratch_types=scratch,
    )
    def gather_kernel(idx_hbm, table_hbm, out_hbm, idx_v, *bufs_and_sems):
        rows = bufs_and_sems[:K]
        gsem = bufs_and_sems[K:2 * K]
        osem = bufs_and_sems[2 * K:]
        wid = lax.axis_index("s") * NC + lax.axis_index("c")
        base = wid * b_per_w
        pltpu.sync_copy(idx_hbm.at[pl.ds(base, b_per_w)], idx_v)

        @pl.loop(0, n_win)
        def _window(w):
            w0 = w * (CH * K)
            gathers = []
            for t in range(K):
                @pl.when(w > 0)
                def _drain():
                    pltpu.make_async_copy(
                        rows[t], out_hbm.at[pl.ds(0, CH)], osem[t]
                    ).wait()

                ids = idx_v.at[pl.ds(w0 + t * CH, CH)]
                gathers.append(
                    pltpu.async_copy(table_hbm.at[ids], rows[t], gsem[t])
                )
            for t in range(K):
                gathers[t].wait()
                pltpu.async_copy(
                    rows[t], out_hbm.at[pl.ds(base + w0 + t * CH, CH)], osem[t]
                )

        for t in range(K):
            pltpu.make_async_copy(
                rows[t], out_hbm.at[pl.ds(0, CH)], osem[t]
            ).wait()

    return gather_kernel


NSLAB = 4


def kernel(x, emb_weight):
    T, S = x.shape
    idx = x.reshape(-1).astype(jnp.int32)
    B = idx.shape[0]
    granule = NW * CH * K
    if T % NSLAB == 0 and (B // NSLAB) % granule == 0:
        fn = _make_gather(B // NSLAB, emb_weight.shape[0])
        ts = T // NSLAB
        outs = [
            fn(lax.dynamic_slice_in_dim(idx, k * (B // NSLAB), B // NSLAB),
               emb_weight).reshape(ts, S, D)
            for k in range(NSLAB)
        ]
        return jnp.concatenate(outs, axis=0)
    B_pad = ((B + granule - 1) // granule) * granule
    if B_pad != B:
        idx = jnp.pad(idx, (0, B_pad - B))
    out = _make_gather(B_pad, emb_weight.shape[0])(idx, emb_weight)
    if B_pad != B:
        out = out[:B]
    return out.reshape(T, S, D)


# speedup vs baseline: 1.0029x; 1.0024x over previous
import functools

import jax
import jax.numpy as jnp
from jax import lax
from jax.experimental import pallas as pl
from jax.experimental.pallas import tpu as pltpu
from jax.experimental.pallas import tpu_sc as plsc

D = 64
NC, NS = 2, 16
NW = NC * NS
CH = 128
K = 8


@functools.lru_cache(maxsize=4)
def _make_gather(B, V):
    b_per_w = B // NW
    n_win = b_per_w // (CH * K)
    mesh = plsc.VectorSubcoreMesh(core_axis_name="c", subcore_axis_name="s")

    scratch = [pltpu.VMEM((b_per_w,), jnp.int32)]
    scratch += [pltpu.VMEM((CH, D), jnp.float32) for _ in range(K)]
    scratch += [pltpu.SemaphoreType.DMA for _ in range(2 * K)]

    @functools.partial(
        pl.kernel,
        out_type=jax.ShapeDtypeStruct((B, D), jnp.float32),
        mesh=mesh,
        compiler_params=pltpu.CompilerParams(use_tc_tiling_on_sc=False),
        scratch_types=scratch,
    )
    def gather_kernel(idx_hbm, table_hbm, out_hbm, idx_v, *bufs_and_sems):
        rows = bufs_and_sems[:K]
        gsem = bufs_and_sems[K:2 * K]
        osem = bufs_and_sems[2 * K:]
        wid = lax.axis_index("s") * NC + lax.axis_index("c")
        base = wid * b_per_w
        pltpu.sync_copy(idx_hbm.at[pl.ds(base, b_per_w)], idx_v)

        @pl.loop(0, n_win)
        def _window(w):
            w0 = w * (CH * K)
            gathers = []
            for t in range(K):
                @pl.when(w > 0)
                def _drain():
                    pltpu.make_async_copy(
                        rows[t], out_hbm.at[pl.ds(0, CH)], osem[t]
                    ).wait()

                ids = idx_v.at[pl.ds(w0 + t * CH, CH)]
                gathers.append(
                    pltpu.async_copy(table_hbm.at[ids], rows[t], gsem[t])
                )
            for t in range(K):
                gathers[t].wait()
                pltpu.async_copy(
                    rows[t], out_hbm.at[pl.ds(base + w0 + t * CH, CH)], osem[t]
                )

        for t in range(K):
            pltpu.make_async_copy(
                rows[t], out_hbm.at[pl.ds(0, CH)], osem[t]
            ).wait()

    return gather_kernel


def kernel(x, emb_weight):
    lead_shape = x.shape
    idx = x.reshape(-1).astype(jnp.int32)
    B = idx.shape[0]
    granule = NW * CH * K
    B_pad = ((B + granule - 1) // granule) * granule
    if B_pad != B:
        idx = jnp.pad(idx, (0, B_pad - B))
    out = _make_gather(B_pad, emb_weight.shape[0])(idx, emb_weight)
    if B_pad != B:
        out = out[:B]
    return out.reshape(*lead_shape, D)
